# baseline (speedup 1.0000x reference)
import functools

import jax
import jax.numpy as jnp
from jax import lax
from jax.experimental import pallas as pl
from jax.experimental.pallas import tpu as pltpu
from jax.experimental.pallas import tpu_sc as plsc

N_NODES = 10000
N_HYPEREDGES = 5000
NNZ = 320000
E_INNER = 640000
D = 128
NUM_CLASSES = 10
NUM_GRAPHS = 16
NUM_LAYERS = 2
INNER_LAYERS = 2

CH = 6400
NCHUNK = NNZ // CH
EB = 128
RPT = CH // 16


def _agg_body(x_hbm, srcs_hbm, dsts_hbm, offs2_hbm, out_hbm,
              acc_sh, offv, idxw, dstw, gidx, sidx, rows, sem):
    cid = lax.axis_index("c")
    sid = lax.axis_index("s")
    iota = lax.iota(jnp.int32, 16)
    for ci in range(NCHUNK // 2):
        c = cid * (NCHUNK // 2) + ci
        base = c * CH
        pltpu.sync_copy(x_hbm.at[pl.ds(base + sid * RPT, RPT)],
                        acc_sh.at[pl.ds(sid * RPT, RPT)])
        pltpu.sync_copy(offs2_hbm.at[c], offv)
        plsc.subcore_barrier()
        v = offv[...]
        lo = v[0]
        hi = v[1]
        per_tile = lax.div(hi - lo + 15, 16)
        e0 = lo + sid * per_tile
        e1 = jnp.minimum(e0 + per_tile, hi)
        nb = jnp.maximum(lax.div(e1 - e0 + (EB - 1), EB), 0)

        def batch_body(j, carry):
            s = e0 + j * EB
            sa = (s // 8) * 8
            sh = s - sa
            pltpu.sync_copy(srcs_hbm.at[pl.ds(sa, EB + 16)], idxw)
            pltpu.sync_copy(dsts_hbm.at[pl.ds(sa, EB + 16)], dstw)
            for j16 in range(EB // 16):
                pos = s + j16 * 16 + iota
                sv = idxw[pl.ds(sh + j16 * 16, 16)]
                dv = dstw[pl.ds(sh + j16 * 16, 16)]
                valid = pos < e1
                gidx[pl.ds(j16 * 16, 16)] = jnp.where(valid, sv, 0)
                sidx[pl.ds(j16 * 16, 16)] = jnp.where(valid, dv - base, CH)
            pltpu.async_copy(x_hbm.at[gidx], rows, sem).wait()
            pltpu.sync_copy(rows, acc_sh.at[sidx], add=True)
            return carry

        lax.fori_loop(0, nb, batch_body, 0)
        plsc.subcore_barrier()
        pltpu.sync_copy(acc_sh.at[pl.ds(sid * RPT, RPT)],
                        out_hbm.at[pl.ds(base + sid * RPT, RPT)])


_sc_agg = pl.kernel(
    _agg_body,
    out_type=jax.ShapeDtypeStruct((NNZ, D), jnp.float32),
    mesh=plsc.VectorSubcoreMesh(core_axis_name="c", subcore_axis_name="s"),
    scratch_types=[
        pltpu.VMEM_SHARED((CH + 8, D), jnp.float32),
        pltpu.VMEM((16,), jnp.int32),
        pltpu.VMEM((EB + 16,), jnp.int32),
        pltpu.VMEM((EB + 16,), jnp.int32),
        pltpu.VMEM((EB,), jnp.int32),
        pltpu.VMEM((EB,), jnp.int32),
        pltpu.VMEM((EB, D), jnp.float32),
        pltpu.SemaphoreType.DMA,
    ],
)


def _edge_prep(edge_index):
    src, dst = edge_index[0], edge_index[1]
    order = jnp.argsort(dst)
    src_s = jnp.take(src, order)
    dst_s = jnp.take(dst, order)
    bounds = jnp.arange(0, NNZ + 1, CH, dtype=jnp.int32)
    offs = jnp.searchsorted(dst_s, bounds).astype(jnp.int32)
    offs2 = jnp.tile(jnp.stack([offs[:-1], offs[1:]], axis=1), (1, 8))
    src_p = jnp.concatenate([src_s, jnp.zeros((256,), jnp.int32)])
    dst_p = jnp.concatenate([dst_s, jnp.zeros((256,), jnp.int32)])
    return src_p, dst_p, offs2


def _mlp_body(h_ref, w1_ref, b1_ref, w2_ref, b2_ref, o_ref):
    h = jnp.maximum(jnp.dot(h_ref[...], w1_ref[...],
                            preferred_element_type=jnp.float32)
                    + b1_ref[...], 0.0)
    o_ref[...] = jnp.maximum(jnp.dot(h, w2_ref[...],
                                     preferred_element_type=jnp.float32)
                             + b2_ref[...], 0.0)


def _gin_mlp(h, w1, b1, w2, b2):
    n = h.shape[0]
    blk = 2000
    row_spec = pl.BlockSpec((blk, D), lambda i: (i, 0))
    w_spec = pl.BlockSpec((D, D), lambda i: (0, 0))
    b_spec = pl.BlockSpec((1, D), lambda i: (0, 0))
    return pl.pallas_call(
        _mlp_body,
        grid=(n // blk,),
        in_specs=[row_spec, w_spec, b_spec, w_spec, b_spec],
        out_specs=row_spec,
        out_shape=jax.ShapeDtypeStruct((n, D), jnp.float32),
    )(h, w1, b1.reshape(1, D), w2, b2.reshape(1, D))


def _emb_body(x_ref, w_ref, b_ref, o_ref):
    o_ref[...] = jnp.dot(x_ref[...], w_ref[...],
                         preferred_element_type=jnp.float32) + b_ref[...]


def _emb(x, w, b):
    n = x.shape[0]
    blk = 2000
    return pl.pallas_call(
        _emb_body,
        grid=(n // blk,),
        in_specs=[pl.BlockSpec((blk, D), lambda i: (i, 0)),
                  pl.BlockSpec((D, D), lambda i: (0, 0)),
                  pl.BlockSpec((1, D), lambda i: (0, 0))],
        out_specs=pl.BlockSpec((blk, D), lambda i: (i, 0)),
        out_shape=jax.ShapeDtypeStruct((n, D), jnp.float32),
    )(x, w, b.reshape(1, D))


def kernel(x_N, W_emb, b_emb, gin_W1, gin_b1, gin_W2, gin_b2, W_pred, b_pred,
           ori_node_idx, node2edge, ori_edge_idx, edge2node,
           edge_index_N, edge_index_E, batch):
    node_x = _emb(x_N, W_emb, b_emb)
    prep_N = _edge_prep(edge_index_N)
    prep_E = _edge_prep(edge_index_E)
    xs = [node_x]
    for l in range(NUM_LAYERS):
        _nx = node_x[ori_node_idx]
        for c in range(INNER_LAYERS):
            idx = l * 4 + c
            h = _sc_agg(_nx, *prep_N)
            _nx = _gin_mlp(h, gin_W1[idx], gin_b1[idx],
                           gin_W2[idx], gin_b2[idx])
        edge_x = jax.nn.relu(jax.ops.segment_sum(_nx, node2edge,
                                                 num_segments=N_HYPEREDGES))
        _ex = edge_x[ori_edge_idx]
        for c in range(INNER_LAYERS):
            idx = l * 4 + 2 + c
            h = _sc_agg(_ex, *prep_E)
            _ex = _gin_mlp(h, gin_W1[idx], gin_b1[idx],
                           gin_W2[idx], gin_b2[idx])
        node_x = jax.nn.relu(jax.ops.segment_sum(_ex, edge2node,
                                                 num_segments=N_NODES))
        xs.append(node_x)
    score = jnp.zeros((NUM_GRAPHS, NUM_CLASSES), jnp.float32)
    for i, x in enumerate(xs):
        pooled = jax.ops.segment_sum(x[ori_node_idx], batch,
                                     num_segments=NUM_GRAPHS)
        score = score + pooled @ W_pred[i] + b_pred[i]
    return score
